# baseline (speedup 1.0000x reference)
import functools

import jax
import jax.numpy as jnp
from jax.experimental import pallas as pl
from jax.experimental.pallas import tpu as pltpu

N = 6144
B = 48
BI = 128
BJ = 128
G = N // BI
R2 = 0.01 * 0.01
NEG_INF = float("-inf")


def _mega_kernel(bounds_ref, segb_ref, pos_ref, posT_ref, batchr_ref,
                 batchc_ref,
                 w1pt1_ref, b11_ref, w2t1_ref, b21_ref,
                 w1xt2_ref, w1pt2_ref, b12_ref, w2t2_ref, b22_ref,
                 w1xt3_ref, w1pt3_ref, b13_ref, w2t3_ref, b23_ref,
                 l1a_ref, l1b_ref, l1c_ref, l1bias_ref,
                 l2t_ref, l2bias_ref, l3t_ref, l3bias_ref,
                 out_ref, x1_ref, x2_ref, x3_ref, flags_ref, bflag_ref):

    WT = 4

    def tile_mask(g, j0, w):
        i0 = g * BI
        posi = pos_ref[pl.ds(i0, BI), 0:3]
        bi = batchr_ref[pl.ds(i0, BI), 0:1]
        row_id = i0 + jax.lax.broadcasted_iota(jnp.int32, (BI, 1), 0)
        sqi = (posi[:, 0:1] * posi[:, 0:1] + posi[:, 1:2] * posi[:, 1:2]
               + posi[:, 2:3] * posi[:, 2:3])
        posjT = posT_ref[:, pl.ds(j0, w)]
        bj = batchc_ref[:, pl.ds(j0, w)]
        col_id = j0 + jax.lax.broadcasted_iota(jnp.int32, (1, w), 1)
        sqj = (posjT[0:1, :] * posjT[0:1, :]
               + posjT[1:2, :] * posjT[1:2, :]
               + posjT[2:3, :] * posjT[2:3, :])
        dot = jnp.dot(posi, posjT, preferred_element_type=jnp.float32)
        d2 = (sqi + sqj) - 2.0 * dot
        return (d2 <= R2) & (bi == bj) & (row_id != col_id)

    def flag_iblock(g, _):
        lo = bounds_ref[g, 0]
        hi = bounds_ref[g, 1]

        def flag_wide(t, anyb):
            c0 = jnp.minimum(lo + WT * t, G - WT)
            mw = tile_mask(g, c0 * BJ, WT * BJ)
            aw = jnp.any(mw).astype(jnp.int32)

            def zero4(_):
                for q in range(WT):
                    flags_ref[g, c0 + q] = 0
                return 0

            def per_sub(_):
                for q in range(WT):
                    flags_ref[g, c0 + q] = jnp.any(
                        mw[:, q * BJ:(q + 1) * BJ]).astype(jnp.int32)
                return 0

            jax.lax.cond(aw != 0, per_sub, zero4, 0)
            return anyb | aw

        nt = (hi - lo + WT - 1) // WT
        anyb = jax.lax.fori_loop(0, nt, flag_wide, jnp.int32(0))
        bflag_ref[g, 0] = anyb
        return 0

    jax.lax.fori_loop(0, G, flag_iblock, 0)
    anyedge = jax.lax.fori_loop(
        0, G, lambda g, a: a | bflag_ref[g, 0], jnp.int32(0))

    def conv_layer(x_src, w1xt_ref, w1pt_ref, b1_ref, w2t_ref, b2_ref,
                   x_dst, out_dim):
        b1 = b1_ref[:, :].reshape(1, 1, -1)

        def iblock_active(g):
            i0 = g * BI
            posi = pos_ref[pl.ds(i0, BI), 0:3]
            bi = batchr_ref[pl.ds(i0, BI), 0:1]
            row_id = i0 + jax.lax.broadcasted_iota(jnp.int32, (BI, 1), 0)
            sqi = (posi[:, 0:1] * posi[:, 0:1] + posi[:, 1:2] * posi[:, 1:2]
                   + posi[:, 2:3] * posi[:, 2:3])

            def body(c, acc):
                j0 = c * BJ

                def heavy(acc):
                    posjT = posT_ref[:, pl.ds(j0, BJ)]
                    bj = batchc_ref[:, pl.ds(j0, BJ)]
                    col_id = j0 + jax.lax.broadcasted_iota(
                        jnp.int32, (1, BJ), 1)
                    sqj = (posjT[0:1, :] * posjT[0:1, :]
                           + posjT[1:2, :] * posjT[1:2, :]
                           + posjT[2:3, :] * posjT[2:3, :])
                    dot = jnp.dot(posi, posjT,
                                  preferred_element_type=jnp.float32)
                    d2 = (sqi + sqj) - 2.0 * dot
                    mask = ((d2 <= R2) & (bi == bj)
                            & (row_id != col_id))
                    relw = b1
                    for k in range(3):
                        relk = posjT[k:k + 1, :] - posi[:, k:k + 1]
                        relw = relw + (relk[:, :, None]
                                       * w1pt_ref[k:k + 1, :][None, :, :])
                    if x_src is not None:
                        xj = x_src[pl.ds(j0, BJ), :]
                        u = jnp.dot(xj, w1xt_ref[:, :],
                                    preferred_element_type=jnp.float32)
                        h1 = jax.nn.relu(u[None, :, :] + relw)
                    else:
                        h1 = jax.nn.relu(relw)
                    hdim = h1.shape[-1]
                    h2 = jnp.dot(h1.reshape(BI * BJ, hdim), w2t_ref[:, :],
                                 preferred_element_type=jnp.float32)
                    h2 = (h2 + b2_ref[:, :]).reshape(BI, BJ, out_dim)
                    penal = jnp.where(mask, 0.0, NEG_INF).astype(jnp.float32)
                    h2 = h2 + penal[:, :, None]
                    return jnp.maximum(acc, jnp.max(h2, axis=1))

                return jax.lax.cond(flags_ref[g, c] != 0, heavy,
                                    lambda a: a, acc)

            acc = jax.lax.fori_loop(
                bounds_ref[g, 0], bounds_ref[g, 1], body,
                jnp.full((BI, out_dim), NEG_INF, jnp.float32))
            x_dst[pl.ds(i0, BI), :] = jnp.maximum(acc, 0.0)
            return 0

        def iblock(g, _):
            def zero(_):
                x_dst[pl.ds(g * BI, BI), :] = jnp.zeros((BI, out_dim),
                                                        jnp.float32)
                return 0

            return jax.lax.cond(bflag_ref[g, 0] != 0,
                                lambda _: iblock_active(g), zero, 0)

        jax.lax.fori_loop(0, G, iblock, 0)

    def seg_body(s, _):
        def blk(b, accs):
            a1, a2, a3 = accs
            mf = (batchr_ref[pl.ds(b * BI, BI), 0:1] == s).astype(jnp.float32)
            a1 = jnp.maximum(a1, jnp.max(x1_ref[pl.ds(b * BI, BI), :] * mf,
                                         axis=0, keepdims=True))
            a2 = jnp.maximum(a2, jnp.max(x2_ref[pl.ds(b * BI, BI), :] * mf,
                                         axis=0, keepdims=True))
            a3 = jnp.maximum(a3, jnp.max(x3_ref[pl.ds(b * BI, BI), :] * mf,
                                         axis=0, keepdims=True))
            return a1, a2, a3

        r1, r2, r3 = jax.lax.fori_loop(
            segb_ref[s, 0], segb_ref[s, 1], blk,
            (jnp.zeros((1, 64), jnp.float32),
             jnp.zeros((1, 128), jnp.float32),
             jnp.zeros((1, 256), jnp.float32)))
        g123 = (jnp.dot(r1, l1a_ref[:, :], preferred_element_type=jnp.float32)
                + jnp.dot(r2, l1b_ref[:, :],
                          preferred_element_type=jnp.float32)
                + jnp.dot(r3, l1c_ref[:, :],
                          preferred_element_type=jnp.float32))
        out_ref[pl.ds(s, 1), :] = g123
        return 0

    def full_tail(_):
        conv_layer(None, None, w1pt1_ref, b11_ref, w2t1_ref, b21_ref,
                   x1_ref, 64)
        conv_layer(x1_ref, w1xt2_ref, w1pt2_ref, b12_ref, w2t2_ref,
                   b22_ref, x2_ref, 128)
        conv_layer(x2_ref, w1xt3_ref, w1pt3_ref, b13_ref, w2t3_ref,
                   b23_ref, x3_ref, 256)
        jax.lax.fori_loop(0, B, seg_body, 0)
        h = jax.nn.relu(out_ref[:, :] + l1bias_ref[:, :])
        h = jax.nn.relu(jnp.dot(h, l2t_ref[:, :],
                                preferred_element_type=jnp.float32)
                        + l2bias_ref[:, :])
        out2 = (jnp.dot(h, l3t_ref[:, :],
                        preferred_element_type=jnp.float32)
                + l3bias_ref[:, :])
        out_ref[:, 0:1] = out2
        return 0

    def fast_tail(_):
        h = jax.nn.relu(jnp.zeros((1, 128), jnp.float32) + l1bias_ref[:, :])
        h = jax.nn.relu(jnp.dot(h, l2t_ref[:, :],
                                preferred_element_type=jnp.float32)
                        + l2bias_ref[:, :])
        out2 = (jnp.dot(h, l3t_ref[:, :],
                        preferred_element_type=jnp.float32)
                + l3bias_ref[:, :])
        out_ref[:, 0:1] = jnp.broadcast_to(out2, (B, 1))
        return 0

    jax.lax.cond(anyedge != 0, full_tail, fast_tail, 0)


def kernel(pos, batch, c1_W1, c1_b1, c1_W2, c1_b2, c2_W1, c2_b1, c2_W2,
           c2_b2, c3_W1, c3_b1, c3_W2, c3_b2, l1_W, l1_b, l2_W, l2_b,
           l3_W, l3_b):
    batch = batch.astype(jnp.int32)
    starts = jnp.searchsorted(batch, jnp.arange(B, dtype=jnp.int32),
                              side="left")
    ends = jnp.searchsorted(batch, jnp.arange(B, dtype=jnp.int32),
                            side="right")
    jlo = starts[batch[::BI]]
    jhi = ends[batch[BI - 1::BI]]
    bounds = jnp.stack([jlo // BJ, (jhi + BJ - 1) // BJ],
                       axis=1).astype(jnp.int32)
    segb = jnp.stack([starts // BI, (ends + BI - 1) // BI],
                     axis=1).astype(jnp.int32)

    posT = pos.T
    posW = jnp.pad(pos, ((0, 0), (0, 125)))
    batchW = jnp.pad(batch.reshape(N, 1), ((0, 0), (0, 127)))
    batchc = batch.reshape(1, N)

    smem = pl.BlockSpec(memory_space=pltpu.SMEM)
    full = lambda a: pl.BlockSpec(a.shape, lambda: (0,) * a.ndim)
    args = (posW, posT, batchW, batchc,
            c1_W1[:, -3:].T, c1_b1.reshape(1, -1), c1_W2.T,
            c1_b2.reshape(1, -1),
            c2_W1[:, :-3].T, c2_W1[:, -3:].T, c2_b1.reshape(1, -1),
            c2_W2.T, c2_b2.reshape(1, -1),
            c3_W1[:, :-3].T, c3_W1[:, -3:].T, c3_b1.reshape(1, -1),
            c3_W2.T, c3_b2.reshape(1, -1),
            l1_W[:, :64].T, l1_W[:, 64:192].T, l1_W[:, 192:].T,
            l1_b.reshape(1, -1), l2_W.T, l2_b.reshape(1, -1),
            l3_W.T, l3_b.reshape(1, -1))
    out = pl.pallas_call(
        _mega_kernel,
        in_specs=[smem, smem] + [full(a) for a in args],
        out_specs=pl.BlockSpec((B, 128), lambda: (0, 0)),
        out_shape=jax.ShapeDtypeStruct((B, 128), jnp.float32),
        scratch_shapes=[pltpu.VMEM((N, 64), jnp.float32),
                        pltpu.VMEM((N, 128), jnp.float32),
                        pltpu.VMEM((N, 256), jnp.float32),
                        pltpu.SMEM((G, G), jnp.int32),
                        pltpu.SMEM((G, 1), jnp.int32)],
    )(bounds, segb, *args)
    return out[:, 0]


# speedup vs baseline: 3.6943x; 3.6943x over previous
import functools

import jax
import jax.numpy as jnp
from jax.experimental import pallas as pl
from jax.experimental.pallas import tpu as pltpu

N = 6144
B = 48
BI = 128
BJ = 128
G = N // BI
R2 = 0.01 * 0.01
NEG_INF = float("-inf")


def _mega_kernel(bounds_ref, segb_ref, pos_ref, posT_ref, batchr_ref,
                 batchc_ref,
                 w1pt1_ref, b11_ref, w2t1_ref, b21_ref,
                 w1xt2_ref, w1pt2_ref, b12_ref, w2t2_ref, b22_ref,
                 w1xt3_ref, w1pt3_ref, b13_ref, w2t3_ref, b23_ref,
                 l1a_ref, l1b_ref, l1c_ref, l1bias_ref,
                 l2t_ref, l2bias_ref, l3t_ref, l3bias_ref,
                 out_ref, x1_ref, x2_ref, x3_ref, flags_ref, bflag_ref):

    WT = 4

    def tile_mask(g, j0, w):
        i0 = g * BI
        posi = pos_ref[pl.ds(i0, BI), 0:3]
        bi = batchr_ref[pl.ds(i0, BI), 0:1]
        row_id = i0 + jax.lax.broadcasted_iota(jnp.int32, (BI, 1), 0)
        sqi = (posi[:, 0:1] * posi[:, 0:1] + posi[:, 1:2] * posi[:, 1:2]
               + posi[:, 2:3] * posi[:, 2:3])
        posjT = posT_ref[:, pl.ds(j0, w)]
        bj = batchc_ref[:, pl.ds(j0, w)]
        col_id = j0 + jax.lax.broadcasted_iota(jnp.int32, (1, w), 1)
        sqj = (posjT[0:1, :] * posjT[0:1, :]
               + posjT[1:2, :] * posjT[1:2, :]
               + posjT[2:3, :] * posjT[2:3, :])
        dot = (posi[:, 0:1] * posjT[0:1, :] + posi[:, 1:2] * posjT[1:2, :]
               + posi[:, 2:3] * posjT[2:3, :])
        d2 = (sqi + sqj) - 2.0 * dot
        return (d2 <= R2) & (bi == bj) & (row_id != col_id)

    def flag_iblock(g, _):
        lo = bounds_ref[g, 0]
        hi = bounds_ref[g, 1]

        def flag_wide(t, anyb):
            c0 = jnp.minimum(lo + WT * t, G - WT)
            mw = tile_mask(g, c0 * BJ, WT * BJ)
            aw = jnp.any(mw).astype(jnp.int32)

            def zero4(_):
                for q in range(WT):
                    flags_ref[g, c0 + q] = 0
                return 0

            def per_sub(_):
                for q in range(WT):
                    flags_ref[g, c0 + q] = jnp.any(
                        mw[:, q * BJ:(q + 1) * BJ]).astype(jnp.int32)
                return 0

            jax.lax.cond(aw != 0, per_sub, zero4, 0)
            return anyb | aw

        nt = (hi - lo + WT - 1) // WT
        anyb = jax.lax.fori_loop(0, nt, flag_wide, jnp.int32(0))
        bflag_ref[g, 0] = anyb
        return 0

    jax.lax.fori_loop(0, G, flag_iblock, 0)
    anyedge = jax.lax.fori_loop(
        0, G, lambda g, a: a | bflag_ref[g, 0], jnp.int32(0))

    def conv_layer(x_src, w1xt_ref, w1pt_ref, b1_ref, w2t_ref, b2_ref,
                   x_dst, out_dim):
        b1 = b1_ref[:, :].reshape(1, 1, -1)

        def iblock_active(g):
            i0 = g * BI
            posi = pos_ref[pl.ds(i0, BI), 0:3]
            bi = batchr_ref[pl.ds(i0, BI), 0:1]
            row_id = i0 + jax.lax.broadcasted_iota(jnp.int32, (BI, 1), 0)
            sqi = (posi[:, 0:1] * posi[:, 0:1] + posi[:, 1:2] * posi[:, 1:2]
                   + posi[:, 2:3] * posi[:, 2:3])

            def body(c, acc):
                j0 = c * BJ

                def heavy(acc):
                    posjT = posT_ref[:, pl.ds(j0, BJ)]
                    bj = batchc_ref[:, pl.ds(j0, BJ)]
                    col_id = j0 + jax.lax.broadcasted_iota(
                        jnp.int32, (1, BJ), 1)
                    sqj = (posjT[0:1, :] * posjT[0:1, :]
                           + posjT[1:2, :] * posjT[1:2, :]
                           + posjT[2:3, :] * posjT[2:3, :])
                    dot = jnp.dot(posi, posjT,
                                  preferred_element_type=jnp.float32)
                    d2 = (sqi + sqj) - 2.0 * dot
                    mask = ((d2 <= R2) & (bi == bj)
                            & (row_id != col_id))
                    relw = b1
                    for k in range(3):
                        relk = posjT[k:k + 1, :] - posi[:, k:k + 1]
                        relw = relw + (relk[:, :, None]
                                       * w1pt_ref[k:k + 1, :][None, :, :])
                    if x_src is not None:
                        xj = x_src[pl.ds(j0, BJ), :]
                        u = jnp.dot(xj, w1xt_ref[:, :],
                                    preferred_element_type=jnp.float32)
                        h1 = jax.nn.relu(u[None, :, :] + relw)
                    else:
                        h1 = jax.nn.relu(relw)
                    hdim = h1.shape[-1]
                    h2 = jnp.dot(h1.reshape(BI * BJ, hdim), w2t_ref[:, :],
                                 preferred_element_type=jnp.float32)
                    h2 = (h2 + b2_ref[:, :]).reshape(BI, BJ, out_dim)
                    penal = jnp.where(mask, 0.0, NEG_INF).astype(jnp.float32)
                    h2 = h2 + penal[:, :, None]
                    return jnp.maximum(acc, jnp.max(h2, axis=1))

                return jax.lax.cond(flags_ref[g, c] != 0, heavy,
                                    lambda a: a, acc)

            acc = jax.lax.fori_loop(
                bounds_ref[g, 0], bounds_ref[g, 1], body,
                jnp.full((BI, out_dim), NEG_INF, jnp.float32))
            x_dst[pl.ds(i0, BI), :] = jnp.maximum(acc, 0.0)
            return 0

        def iblock(g, _):
            def zero(_):
                x_dst[pl.ds(g * BI, BI), :] = jnp.zeros((BI, out_dim),
                                                        jnp.float32)
                return 0

            return jax.lax.cond(bflag_ref[g, 0] != 0,
                                lambda _: iblock_active(g), zero, 0)

        jax.lax.fori_loop(0, G, iblock, 0)

    def seg_body(s, _):
        def blk(b, accs):
            a1, a2, a3 = accs
            mf = (batchr_ref[pl.ds(b * BI, BI), 0:1] == s).astype(jnp.float32)
            a1 = jnp.maximum(a1, jnp.max(x1_ref[pl.ds(b * BI, BI), :] * mf,
                                         axis=0, keepdims=True))
            a2 = jnp.maximum(a2, jnp.max(x2_ref[pl.ds(b * BI, BI), :] * mf,
                                         axis=0, keepdims=True))
            a3 = jnp.maximum(a3, jnp.max(x3_ref[pl.ds(b * BI, BI), :] * mf,
                                         axis=0, keepdims=True))
            return a1, a2, a3

        r1, r2, r3 = jax.lax.fori_loop(
            segb_ref[s, 0], segb_ref[s, 1], blk,
            (jnp.zeros((1, 64), jnp.float32),
             jnp.zeros((1, 128), jnp.float32),
             jnp.zeros((1, 256), jnp.float32)))
        g123 = (jnp.dot(r1, l1a_ref[:, :], preferred_element_type=jnp.float32)
                + jnp.dot(r2, l1b_ref[:, :],
                          preferred_element_type=jnp.float32)
                + jnp.dot(r3, l1c_ref[:, :],
                          preferred_element_type=jnp.float32))
        out_ref[pl.ds(s, 1), :] = g123
        return 0

    def full_tail(_):
        conv_layer(None, None, w1pt1_ref, b11_ref, w2t1_ref, b21_ref,
                   x1_ref, 64)
        conv_layer(x1_ref, w1xt2_ref, w1pt2_ref, b12_ref, w2t2_ref,
                   b22_ref, x2_ref, 128)
        conv_layer(x2_ref, w1xt3_ref, w1pt3_ref, b13_ref, w2t3_ref,
                   b23_ref, x3_ref, 256)
        jax.lax.fori_loop(0, B, seg_body, 0)
        h = jax.nn.relu(out_ref[:, :] + l1bias_ref[:, :])
        h = jax.nn.relu(jnp.dot(h, l2t_ref[:, :],
                                preferred_element_type=jnp.float32)
                        + l2bias_ref[:, :])
        out2 = (jnp.dot(h, l3t_ref[:, :],
                        preferred_element_type=jnp.float32)
                + l3bias_ref[:, :])
        out_ref[:, 0:1] = out2
        return 0

    def fast_tail(_):
        h = jax.nn.relu(jnp.zeros((1, 128), jnp.float32) + l1bias_ref[:, :])
        h = jax.nn.relu(jnp.dot(h, l2t_ref[:, :],
                                preferred_element_type=jnp.float32)
                        + l2bias_ref[:, :])
        out2 = (jnp.dot(h, l3t_ref[:, :],
                        preferred_element_type=jnp.float32)
                + l3bias_ref[:, :])
        out_ref[:, 0:1] = jnp.broadcast_to(out2, (B, 1))
        return 0

    jax.lax.cond(anyedge != 0, full_tail, fast_tail, 0)


def kernel(pos, batch, c1_W1, c1_b1, c1_W2, c1_b2, c2_W1, c2_b1, c2_W2,
           c2_b2, c3_W1, c3_b1, c3_W2, c3_b2, l1_W, l1_b, l2_W, l2_b,
           l3_W, l3_b):
    batch = batch.astype(jnp.int32)
    starts = jnp.searchsorted(batch, jnp.arange(B, dtype=jnp.int32),
                              side="left")
    ends = jnp.searchsorted(batch, jnp.arange(B, dtype=jnp.int32),
                            side="right")
    jlo = starts[batch[::BI]]
    jhi = ends[batch[BI - 1::BI]]
    bounds = jnp.stack([jlo // BJ, (jhi + BJ - 1) // BJ],
                       axis=1).astype(jnp.int32)
    segb = jnp.stack([starts // BI, (ends + BI - 1) // BI],
                     axis=1).astype(jnp.int32)

    posT = pos.T
    posW = jnp.pad(pos, ((0, 0), (0, 125)))
    batchW = jnp.pad(batch.reshape(N, 1), ((0, 0), (0, 127)))
    batchc = batch.reshape(1, N)

    smem = pl.BlockSpec(memory_space=pltpu.SMEM)
    full = lambda a: pl.BlockSpec(a.shape, lambda: (0,) * a.ndim)
    args = (posW, posT, batchW, batchc,
            c1_W1[:, -3:].T, c1_b1.reshape(1, -1), c1_W2.T,
            c1_b2.reshape(1, -1),
            c2_W1[:, :-3].T, c2_W1[:, -3:].T, c2_b1.reshape(1, -1),
            c2_W2.T, c2_b2.reshape(1, -1),
            c3_W1[:, :-3].T, c3_W1[:, -3:].T, c3_b1.reshape(1, -1),
            c3_W2.T, c3_b2.reshape(1, -1),
            l1_W[:, :64].T, l1_W[:, 64:192].T, l1_W[:, 192:].T,
            l1_b.reshape(1, -1), l2_W.T, l2_b.reshape(1, -1),
            l3_W.T, l3_b.reshape(1, -1))
    out = pl.pallas_call(
        _mega_kernel,
        in_specs=[smem, smem] + [full(a) for a in args],
        out_specs=pl.BlockSpec((B, 128), lambda: (0, 0)),
        out_shape=jax.ShapeDtypeStruct((B, 128), jnp.float32),
        scratch_shapes=[pltpu.VMEM((N, 64), jnp.float32),
                        pltpu.VMEM((N, 128), jnp.float32),
                        pltpu.VMEM((N, 256), jnp.float32),
                        pltpu.SMEM((G, G), jnp.int32),
                        pltpu.SMEM((G, 1), jnp.int32)],
    )(bounds, segb, *args)
    return out[:, 0]
